# baseline (speedup 1.0000x reference)
import functools

import jax
import jax.numpy as jnp
from jax import lax
from jax.experimental import pallas as pl
from jax.experimental.pallas import tpu as pltpu
from jax.experimental.pallas import tpu_sc as plsc

B = 4096
F = 26
V = 100000
D = 16

NC = 2
NS = 16
FH = F // NC

CV = 1536
CVT = 32
CVP = CV + CVT
NCH = 66
TAILV = V - CVT
TBASE = TAILV - CV
NTASKC = FH * NCH
TPW = 54

PKH = FH * B // 8
PKТRASH = PKH
PKT = 6784
NZCH = PKT // 128
TRASH_B = B


def _sc_stream_extract(tableT, tailT, xiT):
    mesh = plsc.VectorSubcoreMesh(
        core_axis_name="c", subcore_axis_name="s",
        num_cores=NC, num_subcores=NS)

    @functools.partial(
        pl.kernel,
        mesh=mesh,
        out_type=jax.ShapeDtypeStruct((NC, PKT, 128), jnp.float32),
        compiler_params=pltpu.CompilerParams(
            use_tc_tiling_on_sc=True, needs_layout_passes=False),
        scratch_types=[
            pltpu.VMEM_SHARED((PKT, 128), jnp.float32),
            pltpu.VMEM((2, 16, CVP), jnp.float32),
            pltpu.VMEM((1, B), jnp.int32),
            pltpu.VMEM((B + 32,), jnp.int32),
            pltpu.VMEM((2, 32, 128), jnp.float32),
            pltpu.VMEM((32,), jnp.int32),
            pltpu.VMEM((32,), jnp.int32),
            pltpu.SemaphoreType.DMA,
            pltpu.SemaphoreType.DMA,
            pltpu.SemaphoreType.DMA,
            pltpu.SemaphoreType.DMA,
            pltpu.SemaphoreType.DMA,
        ],
    )
    def k(table_hbm, tail_hbm, xi_hbm, out_hbm,
          es, slab, xirow, seg, q, qidxa, qidxb, sem0, sem1, tsem,
          qsema, qsemb):
        cid = lax.axis_index("c")
        sid = lax.axis_index("s")
        iot = lax.iota(jnp.int32, 16)
        zv = jnp.zeros((16,), jnp.float32)

        zp = jnp.zeros((16,), jnp.int32)
        for r in range(32):
            for cb in range(8):
                plsc.store_scatter(
                    q, [zp, jnp.full((16,), r, jnp.int32), cb * 16 + iot],
                    zv)
        for jz in range(14):
            ch = sid + jz * NS

            @pl.when(ch < PKT // 32)
            def _():
                pltpu.sync_copy(q.at[0], es.at[pl.ds(ch * 32, 32)])
        plsc.subcore_barrier()

        t0 = sid * TPW

        def task_params(t):
            tt = jnp.where(t < NTASKC, t, 0)
            fl = tt // NCH
            c = tt % NCH
            last = c == NCH - 1
            base = jnp.where(last, TBASE, c * CV)
            lo = c * CV
            hi = jnp.where(last, V, (c + 1) * CV)
            hi = jnp.where(t < NTASKC, hi, lo)
            f = cid * FH + fl
            return f, base, lo, hi, last

        def start(t, buf, sem):
            f, base, _, _, last = task_params(t)
            off = pl.multiple_of(base, 128)
            cp = pltpu.async_copy(
                table_hbm.at[f, :, pl.ds(off, CV)],
                slab.at[buf, :, pl.ds(0, CV)], sem)

            @pl.when(last)
            def _():
                pltpu.async_copy(
                    tail_hbm.at[f], slab.at[buf, :, pl.ds(CV, CVT)], tsem)
            return cp

        def process(buf, t, fprev, cnt):
            f, base, lo, hi, last = task_params(t)
            fl = f - cid * FH

            @pl.when(f != fprev)
            def _():
                pltpu.sync_copy(xi_hbm.at[f], xirow)

            @pl.when(last)
            def _():
                pltpu.make_async_copy(
                    tail_hbm.at[0], slab.at[buf, :, pl.ds(CV, CVT)],
                    tsem).wait()

            def scan(kk, scnt):
                vv = xirow[0, pl.ds(kk * 16, 16)]
                m = (vv >= lo) & (vv < hi)
                n = plsc.all_reduce_population_count(m)[0]

                @pl.when(n > 0)
                def _():
                    bb = iot + kk * 16
                    packed = vv | lax.shift_left(bb, 17)
                    plsc.store_compressed(
                        seg.at[pl.ds(scnt, 16)], packed, mask=m)

                return scnt + n

            scnt = lax.fori_loop(0, B // 16, scan, 0)
            plsc.store_compressed(
                seg.at[pl.ds(scnt, 16)],
                jnp.full((16,), lo | (TRASH_B << 17), jnp.int32),
                mask=iot >= 0)
            nblk = (scnt + 15) // 16

            def blk(bi, cnt):
                packed = seg[pl.ds(bi * 16, 16)]
                vvec = packed & 131071
                bvec = lax.shift_right_logical(packed, 17)
                lrow = fl * B + bvec
                lrow = jnp.where(bvec == TRASH_B, FH * B + 64, lrow)
                qrow = lax.shift_right_logical(lrow, 3)
                dcol = (lrow % 8) * 16
                o = cnt % 32
                par = (cnt // 32) % 2

                @pl.when(par == 0)
                def _():
                    qidxa[pl.ds(o, 16)] = qrow

                @pl.when(par == 1)
                def _():
                    qidxb[pl.ds(o, 16)] = qrow

                pvec = jnp.full((16,), par, jnp.int32)
                for jj in range(16):
                    cols = jnp.full((16,), vvec[jj] - base, jnp.int32)
                    val = plsc.load_gather(slab.at[buf], [iot, cols])
                    slot = jnp.full((16,), o + jj, jnp.int32)
                    dc = dcol[jj]
                    for cb in range(8):
                        vsel = jnp.where(dc == cb * 16, val, zv)
                        plsc.store_scatter(
                            q, [pvec, slot, cb * 16 + iot], vsel)
                cnt = cnt + 16

                @pl.when(cnt % 32 == 0)
                def _():
                    fpar = (cnt // 32 - 1) % 2

                    @pl.when(fpar == 0)
                    def _():
                        @pl.when(cnt >= 96)
                        def _():
                            pltpu.make_async_copy(
                                q.at[0], es.at[qidxa], qsema).wait()
                        pltpu.async_copy(q.at[0], es.at[qidxa], qsema,
                                         add=True)

                    @pl.when(fpar == 1)
                    def _():
                        @pl.when(cnt >= 96)
                        def _():
                            pltpu.make_async_copy(
                                q.at[1], es.at[qidxb], qsemb).wait()
                        pltpu.async_copy(q.at[1], es.at[qidxb], qsemb,
                                         add=True)

                return cnt

            cnt = lax.fori_loop(0, nblk, blk, cnt)
            return f, cnt

        start(t0, 0, sem0)
        start(t0 + 1, 1, sem1)

        def pair(p, carry):
            fprev, cnt = carry
            tA = t0 + 2 * p
            pltpu.make_async_copy(
                table_hbm.at[0, :, pl.ds(0, CV)],
                slab.at[0, :, pl.ds(0, CV)], sem0).wait()
            fprev, cnt = process(0, tA, fprev, cnt)

            @pl.when(p < TPW // 2 - 1)
            def _():
                start(tA + 2, 0, sem0)

            pltpu.make_async_copy(
                table_hbm.at[0, :, pl.ds(0, CV)],
                slab.at[1, :, pl.ds(0, CV)], sem1).wait()
            fprev, cnt = process(1, tA + 1, fprev, cnt)

            @pl.when(p < TPW // 2 - 1)
            def _():
                start(tA + 3, 1, sem1)

            return fprev, cnt

        _, cnt = lax.fori_loop(
            0, TPW // 2, pair, (jnp.int32(-1), jnp.int32(0)))

        nfire = cnt // 32

        @pl.when(nfire >= 1)
        def _():
            lpar = (nfire - 1) % 2

            @pl.when(lpar == 0)
            def _():
                pltpu.make_async_copy(q.at[0], es.at[qidxa], qsema).wait()

            @pl.when(lpar == 1)
            def _():
                pltpu.make_async_copy(q.at[1], es.at[qidxb], qsemb).wait()

        @pl.when(nfire >= 2)
        def _():
            ppar = (nfire - 2) % 2

            @pl.when(ppar == 0)
            def _():
                pltpu.make_async_copy(q.at[0], es.at[qidxa], qsema).wait()

            @pl.when(ppar == 1)
            def _():
                pltpu.make_async_copy(q.at[1], es.at[qidxb], qsemb).wait()

        o = cnt % 32
        fpar = (cnt // 32) % 2
        trash = jnp.full((16,), PKH + 32, jnp.int32)
        for bi in range(2):
            @pl.when(bi * 16 >= o)
            def _():
                @pl.when(fpar == 0)
                def _():
                    qidxa[pl.ds(bi * 16, 16)] = trash

                @pl.when(fpar == 1)
                def _():
                    qidxb[pl.ds(bi * 16, 16)] = trash

        @pl.when(o > 0)
        def _():
            @pl.when(fpar == 0)
            def _():
                pltpu.sync_copy(q.at[0], es.at[qidxa], add=True)

            @pl.when(fpar == 1)
            def _():
                pltpu.sync_copy(q.at[1], es.at[qidxb], add=True)

        plsc.subcore_barrier()
        for jz in range(4):
            ch = sid + jz * NS

            @pl.when(ch < NZCH)
            def _():
                pltpu.sync_copy(
                    es.at[pl.ds(ch * 128, 128)],
                    out_hbm.at[cid, pl.ds(ch * 128, 128)])

    return k(tableT, tailT, xiT)


def _tc_body(e_ref, xv_ref, s_ref, t_ref, w1_ref, b1_ref, g1_ref, be1_ref,
             w2_ref, b2_ref, g2_ref, be2_ref, bias_ref, out_ref):
    f32 = jnp.float32
    xw = jax.lax.dot_general(xv_ref[...], s_ref[...], (((1,), (0,)), ((), ())),
                             preferred_element_type=f32)
    arr = e_ref[...] * xw

    s = jax.lax.dot_general(arr, t_ref[...], (((1,), (0,)), ((), ())),
                            preferred_element_type=f32)
    ssq = jax.lax.dot_general(arr * arr, t_ref[...], (((1,), (0,)), ((), ())),
                              preferred_element_type=f32)
    fm = 0.5 * (s * s - ssq)

    x1 = jax.lax.dot_general(arr, w1_ref[...], (((1,), (0,)), ((), ())),
                             preferred_element_type=f32) + b1_ref[...]
    m1 = jnp.mean(x1, axis=0, keepdims=True)
    v1 = jnp.mean((x1 - m1) ** 2, axis=0, keepdims=True)
    h1 = g1_ref[...] * (x1 - m1) * lax.rsqrt(v1 + 1e-5) + be1_ref[...]

    x2 = jax.lax.dot_general(h1, w2_ref[...], (((1,), (0,)), ((), ())),
                             preferred_element_type=f32) + b2_ref[...]
    m2 = jnp.mean(x2, axis=0, keepdims=True)
    v2 = jnp.mean((x2 - m2) ** 2, axis=0, keepdims=True)
    h2 = g2_ref[...] * (x2 - m2) * lax.rsqrt(v2 + 1e-5) + be2_ref[...]

    total = (jnp.sum(fm, axis=1, keepdims=True)
             + jnp.sum(h2, axis=1, keepdims=True)
             + bias_ref[...])
    p = 1.0 / (1.0 + jnp.exp(-total))
    out_ref[...] = jnp.clip(p, 0.005, 0.995)


def kernel(Xi, Xv, emb, W1, b1, g1, be1, W2, b2, g2, be2, bias):
    embT = emb.transpose(0, 2, 1)
    tailT = embT[:, :, TAILV:]
    xiT = Xi[:, :, 0].astype(jnp.int32).T.reshape(F, 1, B)

    eh = _sc_stream_extract(embT, tailT, xiT)
    ef = jnp.concatenate([eh[0, :PKH], eh[1, :PKH]], axis=0)
    e2 = ef.reshape(F, B, D).transpose(1, 0, 2).reshape(B, F * D)

    S = jnp.repeat(jnp.eye(F, dtype=jnp.float32), D, axis=1)
    T = jnp.tile(jnp.eye(D, dtype=jnp.float32), (F, 1))

    out = pl.pallas_call(
        _tc_body,
        out_shape=jax.ShapeDtypeStruct((B, 1), jnp.float32),
    )(e2, Xv, S, T, W1, b1.reshape(1, 128), g1.reshape(1, 128),
      be1.reshape(1, 128), W2, b2.reshape(1, 128), g2.reshape(1, 128),
      be2.reshape(1, 128), bias.reshape(B, 1))
    return out.reshape(B)


# speedup vs baseline: 1.3547x; 1.3547x over previous
import functools

import jax
import jax.numpy as jnp
from jax import lax
from jax.experimental import pallas as pl
from jax.experimental.pallas import tpu as pltpu
from jax.experimental.pallas import tpu_sc as plsc

B = 4096
F = 26
V = 100000
D = 16

NC = 2
NS = 16
FH = F // NC

CV = 1536
CVT = 32
CVP = CV + CVT
NCH = 66
TAILV = V - CVT
TBASE = TAILV - CV
NTASKC = FH * NCH
TPW = 54

PKH = FH * B // 8
PKT = 6784
NZCH = PKT // 128
TRASH_B = B


def _sc_stream_extract(tableT, tailT, xiT):
    mesh = plsc.VectorSubcoreMesh(
        core_axis_name="c", subcore_axis_name="s",
        num_cores=NC, num_subcores=NS)

    @functools.partial(
        pl.kernel,
        mesh=mesh,
        out_type=jax.ShapeDtypeStruct((NC, PKT, 128), jnp.float32),
        compiler_params=pltpu.CompilerParams(
            use_tc_tiling_on_sc=True, needs_layout_passes=False),
        scratch_types=[
            pltpu.VMEM_SHARED((PKT, 128), jnp.float32),
            pltpu.VMEM((2, 16, CVP), jnp.float32),
            pltpu.VMEM((1, B), jnp.int32),
            pltpu.VMEM((B + 32,), jnp.int32),
            pltpu.VMEM((2, 32, 128), jnp.float32),
            pltpu.VMEM((32,), jnp.int32),
            pltpu.VMEM((32,), jnp.int32),
            pltpu.SemaphoreType.DMA,
            pltpu.SemaphoreType.DMA,
            pltpu.SemaphoreType.DMA,
            pltpu.SemaphoreType.DMA,
            pltpu.SemaphoreType.DMA,
        ],
    )
    def k(table_hbm, tail_hbm, xi_hbm, out_hbm,
          es, slab, xirow, seg, q, qidxa, qidxb, sem0, sem1, tsem,
          qsema, qsemb):
        cid = lax.axis_index("c")
        sid = lax.axis_index("s")
        iot = lax.iota(jnp.int32, 16)
        zv = jnp.zeros((16,), jnp.float32)

        zp = jnp.zeros((16,), jnp.int32)
        for r in range(32):
            for cb in range(8):
                plsc.store_scatter(
                    q, [zp, jnp.full((16,), r, jnp.int32), cb * 16 + iot],
                    zv)
        for jz in range(14):
            ch = sid + jz * NS

            @pl.when(ch < PKT // 32)
            def _():
                pltpu.sync_copy(q.at[0], es.at[pl.ds(ch * 32, 32)])
        plsc.subcore_barrier()

        t0 = sid * TPW

        def task_params(t):
            tt = jnp.where(t < NTASKC, t, 0)
            fl = tt // NCH
            c = tt % NCH
            last = c == NCH - 1
            base = jnp.where(last, TBASE, c * CV)
            lo = c * CV
            hi = jnp.where(last, V, (c + 1) * CV)
            hi = jnp.where(t < NTASKC, hi, lo)
            f = cid * FH + fl
            return f, base, lo, hi, last

        def start(t, buf, sem):
            f, base, _, _, last = task_params(t)
            off = pl.multiple_of(base, 128)
            cp = pltpu.async_copy(
                table_hbm.at[f, :, pl.ds(off, CV)],
                slab.at[buf, :, pl.ds(0, CV)], sem)

            @pl.when(last)
            def _():
                pltpu.async_copy(
                    tail_hbm.at[f], slab.at[buf, :, pl.ds(CV, CVT)], tsem)
            return cp

        def process(buf, t, fprev, cnt):
            f, base, lo, hi, last = task_params(t)
            fl = f - cid * FH

            @pl.when(f != fprev)
            def _():
                pltpu.sync_copy(xi_hbm.at[f], xirow)

            @pl.when(last)
            def _():
                pltpu.make_async_copy(
                    tail_hbm.at[0], slab.at[buf, :, pl.ds(CV, CVT)],
                    tsem).wait()

            def scan(kk, scnt):
                for u in range(2):
                    vv = xirow[0, pl.ds(kk * 32 + u * 16, 16)]
                    m = (vv >= lo) & (vv < hi)
                    bb = iot + (kk * 32 + u * 16)
                    packed = vv | lax.shift_left(bb, 17)
                    plsc.store_compressed(
                        seg.at[pl.ds(scnt, 16)], packed, mask=m)
                    scnt = scnt + plsc.all_reduce_population_count(m)[0]
                return scnt

            scnt = lax.fori_loop(0, B // 32, scan, 0)
            plsc.store_compressed(
                seg.at[pl.ds(scnt, 16)],
                jnp.full((16,), lo | (TRASH_B << 17), jnp.int32),
                mask=iot >= 0)
            nblk = (scnt + 15) // 16

            def blk(bi, cnt):
                packed = seg[pl.ds(bi * 16, 16)]
                vvec = packed & 131071
                bvec = lax.shift_right_logical(packed, 17)
                lrow = fl * B + bvec
                lrow = jnp.where(bvec == TRASH_B, FH * B + 64, lrow)
                qrow = lax.shift_right_logical(lrow, 3)
                dcol = (lrow % 8) * 16
                o = cnt % 32
                par = (cnt // 32) % 2

                @pl.when(par == 0)
                def _():
                    qidxa[pl.ds(o, 16)] = qrow

                @pl.when(par == 1)
                def _():
                    qidxb[pl.ds(o, 16)] = qrow

                pvec = jnp.full((16,), par, jnp.int32)
                for jj in range(16):
                    cols = jnp.full((16,), vvec[jj] - base, jnp.int32)
                    val = plsc.load_gather(slab.at[buf], [iot, cols])
                    slot = jnp.full((16,), o + jj, jnp.int32)
                    dc = dcol[jj]
                    for cb in range(8):
                        vsel = jnp.where(dc == cb * 16, val, zv)
                        plsc.store_scatter(
                            q, [pvec, slot, cb * 16 + iot], vsel)
                cnt = cnt + 16

                @pl.when(cnt % 32 == 0)
                def _():
                    fpar = (cnt // 32 - 1) % 2

                    @pl.when(fpar == 0)
                    def _():
                        @pl.when(cnt >= 96)
                        def _():
                            pltpu.make_async_copy(
                                q.at[0], es.at[qidxa], qsema).wait()
                        pltpu.async_copy(q.at[0], es.at[qidxa], qsema,
                                         add=True)

                    @pl.when(fpar == 1)
                    def _():
                        @pl.when(cnt >= 96)
                        def _():
                            pltpu.make_async_copy(
                                q.at[1], es.at[qidxb], qsemb).wait()
                        pltpu.async_copy(q.at[1], es.at[qidxb], qsemb,
                                         add=True)

                return cnt

            cnt = lax.fori_loop(0, nblk, blk, cnt)
            return f, cnt

        start(t0, 0, sem0)
        start(t0 + 1, 1, sem1)

        def pair(p, carry):
            fprev, cnt = carry
            tA = t0 + 2 * p
            pltpu.make_async_copy(
                table_hbm.at[0, :, pl.ds(0, CV)],
                slab.at[0, :, pl.ds(0, CV)], sem0).wait()
            fprev, cnt = process(0, tA, fprev, cnt)

            @pl.when(p < TPW // 2 - 1)
            def _():
                start(tA + 2, 0, sem0)

            pltpu.make_async_copy(
                table_hbm.at[0, :, pl.ds(0, CV)],
                slab.at[1, :, pl.ds(0, CV)], sem1).wait()
            fprev, cnt = process(1, tA + 1, fprev, cnt)

            @pl.when(p < TPW // 2 - 1)
            def _():
                start(tA + 3, 1, sem1)

            return fprev, cnt

        _, cnt = lax.fori_loop(
            0, TPW // 2, pair, (jnp.int32(-1), jnp.int32(0)))

        nfire = cnt // 32

        @pl.when(nfire >= 1)
        def _():
            lpar = (nfire - 1) % 2

            @pl.when(lpar == 0)
            def _():
                pltpu.make_async_copy(q.at[0], es.at[qidxa], qsema).wait()

            @pl.when(lpar == 1)
            def _():
                pltpu.make_async_copy(q.at[1], es.at[qidxb], qsemb).wait()

        @pl.when(nfire >= 2)
        def _():
            ppar = (nfire - 2) % 2

            @pl.when(ppar == 0)
            def _():
                pltpu.make_async_copy(q.at[0], es.at[qidxa], qsema).wait()

            @pl.when(ppar == 1)
            def _():
                pltpu.make_async_copy(q.at[1], es.at[qidxb], qsemb).wait()

        o = cnt % 32
        fpar = (cnt // 32) % 2
        trash = jnp.full((16,), PKH + 32, jnp.int32)
        for bi in range(2):
            @pl.when(bi * 16 >= o)
            def _():
                @pl.when(fpar == 0)
                def _():
                    qidxa[pl.ds(bi * 16, 16)] = trash

                @pl.when(fpar == 1)
                def _():
                    qidxb[pl.ds(bi * 16, 16)] = trash

        @pl.when(o > 0)
        def _():
            @pl.when(fpar == 0)
            def _():
                pltpu.sync_copy(q.at[0], es.at[qidxa], add=True)

            @pl.when(fpar == 1)
            def _():
                pltpu.sync_copy(q.at[1], es.at[qidxb], add=True)

        plsc.subcore_barrier()
        for jz in range(4):
            ch = sid + jz * NS

            @pl.when(ch < NZCH)
            def _():
                pltpu.sync_copy(
                    es.at[pl.ds(ch * 128, 128)],
                    out_hbm.at[cid, pl.ds(ch * 128, 128)])

    return k(tableT, tailT, xiT)


def _tc_body(e_ref, xv_ref, s_ref, t_ref, w1_ref, b1_ref, g1_ref, be1_ref,
             w2_ref, b2_ref, g2_ref, be2_ref, bias_ref, out_ref):
    f32 = jnp.float32
    xw = jax.lax.dot_general(xv_ref[...], s_ref[...], (((1,), (0,)), ((), ())),
                             preferred_element_type=f32)
    arr = e_ref[...] * xw

    s = jax.lax.dot_general(arr, t_ref[...], (((1,), (0,)), ((), ())),
                            preferred_element_type=f32)
    ssq = jax.lax.dot_general(arr * arr, t_ref[...], (((1,), (0,)), ((), ())),
                              preferred_element_type=f32)
    fm = 0.5 * (s * s - ssq)

    x1 = jax.lax.dot_general(arr, w1_ref[...], (((1,), (0,)), ((), ())),
                             preferred_element_type=f32) + b1_ref[...]
    m1 = jnp.mean(x1, axis=0, keepdims=True)
    v1 = jnp.mean((x1 - m1) ** 2, axis=0, keepdims=True)
    h1 = g1_ref[...] * (x1 - m1) * lax.rsqrt(v1 + 1e-5) + be1_ref[...]

    x2 = jax.lax.dot_general(h1, w2_ref[...], (((1,), (0,)), ((), ())),
                             preferred_element_type=f32) + b2_ref[...]
    m2 = jnp.mean(x2, axis=0, keepdims=True)
    v2 = jnp.mean((x2 - m2) ** 2, axis=0, keepdims=True)
    h2 = g2_ref[...] * (x2 - m2) * lax.rsqrt(v2 + 1e-5) + be2_ref[...]

    total = (jnp.sum(fm, axis=1, keepdims=True)
             + jnp.sum(h2, axis=1, keepdims=True)
             + bias_ref[...])
    p = 1.0 / (1.0 + jnp.exp(-total))
    out_ref[...] = jnp.clip(p, 0.005, 0.995)


def kernel(Xi, Xv, emb, W1, b1, g1, be1, W2, b2, g2, be2, bias):
    embT = emb.transpose(0, 2, 1)
    tailT = embT[:, :, TAILV:]
    xiT = Xi[:, :, 0].astype(jnp.int32).T.reshape(F, 1, B)

    eh = _sc_stream_extract(embT, tailT, xiT)
    ef = jnp.concatenate([eh[0, :PKH], eh[1, :PKH]], axis=0)
    e2 = ef.reshape(F, B, D).transpose(1, 0, 2).reshape(B, F * D)

    S = jnp.repeat(jnp.eye(F, dtype=jnp.float32), D, axis=1)
    T = jnp.tile(jnp.eye(D, dtype=jnp.float32), (F, 1))

    out = pl.pallas_call(
        _tc_body,
        out_shape=jax.ShapeDtypeStruct((B, 1), jnp.float32),
    )(e2, Xv, S, T, W1, b1.reshape(1, 128), g1.reshape(1, 128),
      be1.reshape(1, 128), W2, b2.reshape(1, 128), g2.reshape(1, 128),
      be2.reshape(1, 128), bias.reshape(B, 1))
    return out.reshape(B)
